# baseline (speedup 1.0000x reference)
import functools

import jax
import jax.numpy as jnp
from jax import lax
from jax.experimental import pallas as pl
from jax.experimental.pallas import tpu as pltpu
from jax.experimental.pallas import tpu_sc as plsc

EMB = 64
VOCAB = 1000000
NI, NJ = 4096, 200
TOTAL = NI * NJ
NUM_WORKERS = 32
UNIT = 2048
N_UNITS = TOTAL // UNIT
UPW = (N_UNITS + NUM_WORKERS - 1) // NUM_WORKERS
CHUNK = 512
CPU_ = UNIT // CHUNK
NBUF = 2

_mesh = plsc.VectorSubcoreMesh(core_axis_name="c", subcore_axis_name="s")


def _eye64():
    r = lax.broadcasted_iota(jnp.int32, (EMB, EMB), 0)
    c = lax.broadcasted_iota(jnp.int32, (EMB, EMB), 1)
    return jnp.where(r == c, 1.0, 0.0).astype(jnp.float32)


def _pack_body(i_ref, o_ref):
    t = lax.dot_general(i_ref[...], _eye64(), (((0,), (0,)), ((), ())),
                        preferred_element_type=jnp.float32)
    t3 = t.reshape(1024, 2, EMB)
    o_ref[:, 0:EMB] = t3[:, 0, :]
    o_ref[:, EMB:128] = t3[:, 1, :]


_tc_pack = pl.pallas_call(
    _pack_body,
    grid=(489,),
    in_specs=[pl.BlockSpec((EMB, UNIT), lambda b: (0, b))],
    out_specs=pl.BlockSpec((1024, 128), lambda b: (b, 0)),
    out_shape=jax.ShapeDtypeStruct((VOCAB // 2, 128), jnp.float32),
)


@functools.partial(
    pl.kernel,
    mesh=_mesh,
    compiler_params=pltpu.CompilerParams(use_tc_tiling_on_sc=False),
    out_type=jax.ShapeDtypeStruct((TOTAL // 2, 128), jnp.float32),
    scratch_types=[
        pltpu.VMEM((UPW * UNIT,), jnp.int32),
        pltpu.VMEM((CHUNK, EMB), jnp.float32),
        pltpu.VMEM((CHUNK, EMB), jnp.float32),
        pltpu.SemaphoreType.DMA,
        pltpu.SemaphoreType.DMA,
        pltpu.SemaphoreType.DMA,
        pltpu.SemaphoreType.DMA,
    ],
)
def _gather_all(idx_hbm, table_hbm, out_hbm, idx_v, rows0, rows1,
                sg0, sg1, ss0, ss1):
    wid = lax.axis_index("s") * 2 + lax.axis_index("c")

    rows = (rows0, rows1)
    sg = (sg0, sg1)
    ss = (ss0, ss1)

    def unit_id(t):
        return t * NUM_WORKERS + wid

    for t in range(UPW):
        u = unit_id(t)

        @pl.when(u < N_UNITS)
        def _(t=t, u=u):
            pltpu.sync_copy(idx_hbm.at[pl.ds(u * UNIT, UNIT)],
                            idx_v.at[pl.ds(t * UNIT, UNIT)])

    NK = UPW * CPU_

    def start_gather(k):
        t, c, b = k // CPU_, k % CPU_, k % NBUF

        @pl.when(unit_id(t) < N_UNITS)
        def _():
            pltpu.async_copy(
                table_hbm.at[idx_v.at[pl.ds(t * UNIT + c * CHUNK, CHUNK)]],
                rows[b], sg[b])

    def wait_gather(k):
        t, c, b = k // CPU_, k % CPU_, k % NBUF

        @pl.when(unit_id(t) < N_UNITS)
        def _():
            pltpu.make_async_copy(
                table_hbm.at[idx_v.at[pl.ds(0, CHUNK)]], rows[b], sg[b]).wait()

    def start_store(k):
        t, c, b = k // CPU_, k % CPU_, k % NBUF
        u = unit_id(t)

        @pl.when(u < N_UNITS)
        def _():
            j = u // 2
            h = u % 2
            pltpu.async_copy(
                rows[b],
                out_hbm.at[pl.ds(j * UNIT + c * CHUNK, CHUNK),
                           pl.ds(h * EMB, EMB)], ss[b])

    def wait_store(k):
        t, c, b = k // CPU_, k % CPU_, k % NBUF

        @pl.when(unit_id(t) < N_UNITS)
        def _():
            pltpu.make_async_copy(
                rows[b], out_hbm.at[pl.ds(0, CHUNK), pl.ds(0, EMB)],
                ss[b]).wait()

    start_gather(0)
    for k in range(NK):
        if k + 1 < NK:
            if k >= 1:
                wait_store(k - 1)
            start_gather(k + 1)
        wait_gather(k)
        start_store(k)
    wait_store(NK - 2)
    wait_store(NK - 1)


def _out_xpose_body(i_ref, o_ref):
    blk = i_ref[...]
    dn = (((1,), (1,)), ((), ()))
    o_ref[0, :, 0:NI // 2] = lax.dot_general(
        _eye64(), blk[:, 0:EMB], dn, preferred_element_type=jnp.float32)
    o_ref[0, :, NI // 2:NI] = lax.dot_general(
        _eye64(), blk[:, EMB:128], dn, preferred_element_type=jnp.float32)


_out_xpose = pl.pallas_call(
    _out_xpose_body,
    grid=(NJ,),
    in_specs=[pl.BlockSpec((NI // 2, 128), lambda j: (j, 0))],
    out_specs=pl.BlockSpec((1, EMB, NI), lambda j: (j, 0, 0)),
    out_shape=jax.ShapeDtypeStruct((NJ, EMB, NI), jnp.float32),
)


def kernel(x, table):
    rm = _tc_pack(table.T).reshape(VOCAB, EMB)
    flat = x.T.reshape(TOTAL)
    out = _gather_all(flat, rm)
    o3 = _out_xpose(out)
    return o3.transpose(2, 0, 1)


# speedup vs baseline: 1.2714x; 1.2714x over previous
import functools

import jax
import jax.numpy as jnp
from jax import lax
from jax.experimental import pallas as pl
from jax.experimental.pallas import tpu as pltpu
from jax.experimental.pallas import tpu_sc as plsc

EMB = 64
VOCAB = 1000000
NI, NJ = 4096, 200
TOTAL = NI * NJ
NUM_WORKERS = 32
UNIT = 2048
N_UNITS = TOTAL // UNIT
UPW = (N_UNITS + NUM_WORKERS - 1) // NUM_WORKERS
CHUNK = 512
CPU_ = UNIT // CHUNK
NBUF = 2

_mesh = plsc.VectorSubcoreMesh(core_axis_name="c", subcore_axis_name="s")


def _eye64():
    r = lax.broadcasted_iota(jnp.int32, (EMB, EMB), 0)
    c = lax.broadcasted_iota(jnp.int32, (EMB, EMB), 1)
    return jnp.where(r == c, 1.0, 0.0).astype(jnp.float32)


def _pack_body(i_ref, o_ref):
    t = lax.dot_general(i_ref[...], _eye64(), (((0,), (0,)), ((), ())),
                        preferred_element_type=jnp.float32)
    t3 = t.reshape(8192, 2, EMB)
    o_ref[:, 0:EMB] = t3[:, 0, :]
    o_ref[:, EMB:128] = t3[:, 1, :]


_tc_pack = pl.pallas_call(
    _pack_body,
    grid=(62,),
    in_specs=[pl.BlockSpec((EMB, 16384), lambda b: (0, b))],
    out_specs=pl.BlockSpec((8192, 128), lambda b: (b, 0)),
    out_shape=jax.ShapeDtypeStruct((VOCAB // 2, 128), jnp.float32),
)


@functools.partial(
    pl.kernel,
    mesh=_mesh,
    compiler_params=pltpu.CompilerParams(use_tc_tiling_on_sc=False),
    out_type=jax.ShapeDtypeStruct((TOTAL // 2, 128), jnp.float32),
    scratch_types=[
        pltpu.VMEM((UPW * UNIT,), jnp.int32),
        pltpu.VMEM((CHUNK, EMB), jnp.float32),
        pltpu.VMEM((CHUNK, EMB), jnp.float32),
        pltpu.SemaphoreType.DMA,
        pltpu.SemaphoreType.DMA,
        pltpu.SemaphoreType.DMA,
        pltpu.SemaphoreType.DMA,
    ],
)
def _gather_all(idx_hbm, table_hbm, out_hbm, idx_v, rows0, rows1,
                sg0, sg1, ss0, ss1):
    wid = lax.axis_index("s") * 2 + lax.axis_index("c")

    rows = (rows0, rows1)
    sg = (sg0, sg1)
    ss = (ss0, ss1)

    def unit_id(t):
        return t * NUM_WORKERS + wid

    for t in range(UPW):
        u = unit_id(t)

        @pl.when(u < N_UNITS)
        def _(t=t, u=u):
            pltpu.sync_copy(idx_hbm.at[pl.ds(u * UNIT, UNIT)],
                            idx_v.at[pl.ds(t * UNIT, UNIT)])

    NK = UPW * CPU_

    def start_gather(k):
        t, c, b = k // CPU_, k % CPU_, k % NBUF

        @pl.when(unit_id(t) < N_UNITS)
        def _():
            pltpu.async_copy(
                table_hbm.at[idx_v.at[pl.ds(t * UNIT + c * CHUNK, CHUNK)]],
                rows[b], sg[b])

    def wait_gather(k):
        t, c, b = k // CPU_, k % CPU_, k % NBUF

        @pl.when(unit_id(t) < N_UNITS)
        def _():
            pltpu.make_async_copy(
                table_hbm.at[idx_v.at[pl.ds(0, CHUNK)]], rows[b], sg[b]).wait()

    def start_store(k):
        t, c, b = k // CPU_, k % CPU_, k % NBUF
        u = unit_id(t)

        @pl.when(u < N_UNITS)
        def _():
            j = u // 2
            h = u % 2
            pltpu.async_copy(
                rows[b],
                out_hbm.at[pl.ds(j * UNIT + c * CHUNK, CHUNK),
                           pl.ds(h * EMB, EMB)], ss[b])

    def wait_store(k):
        t, c, b = k // CPU_, k % CPU_, k % NBUF

        @pl.when(unit_id(t) < N_UNITS)
        def _():
            pltpu.make_async_copy(
                rows[b], out_hbm.at[pl.ds(0, CHUNK), pl.ds(0, EMB)],
                ss[b]).wait()

    start_gather(0)
    for k in range(NK):
        if k + 1 < NK:
            if k >= 1:
                wait_store(k - 1)
            start_gather(k + 1)
        wait_gather(k)
        start_store(k)
    wait_store(NK - 2)
    wait_store(NK - 1)


def _out_xpose_body(i_ref, o_ref):
    blk = i_ref[...]
    dn = (((1,), (1,)), ((), ()))
    for jj in (0, 1):
        sub = blk[jj * 2048:(jj + 1) * 2048, :]
        o_ref[jj, :, 0:NI // 2] = lax.dot_general(
            _eye64(), sub[:, 0:EMB], dn, preferred_element_type=jnp.float32)
        o_ref[jj, :, NI // 2:NI] = lax.dot_general(
            _eye64(), sub[:, EMB:128], dn, preferred_element_type=jnp.float32)


_out_xpose = pl.pallas_call(
    _out_xpose_body,
    grid=(NJ // 2,),
    in_specs=[pl.BlockSpec((NI, 128), lambda g: (g, 0))],
    out_specs=pl.BlockSpec((2, EMB, NI), lambda g: (g, 0, 0)),
    out_shape=jax.ShapeDtypeStruct((NJ, EMB, NI), jnp.float32),
)


def kernel(x, table):
    rm = _tc_pack(table.T).reshape(VOCAB, EMB)
    flat = x.T.reshape(TOTAL)
    out = _gather_all(flat, rm)
    o3 = _out_xpose(out)
    return o3.transpose(2, 0, 1)
